# Initial kernel scaffold; baseline (speedup 1.0000x reference)
import jax
import jax.numpy as jnp
from jax.experimental import pallas as pl


def kernel(strong_preds, ground_truths):
    raise NotImplementedError("write your pallas kernel here")


# speedup vs baseline: 249.9152x; 249.9152x over previous
import functools

import jax
import jax.numpy as jnp
from jax.experimental import pallas as pl
from jax.experimental.pallas import tpu as pltpu


def _sed_f1_kernel(p_ref, l_ref, out_ref):
    p = p_ref[...]
    l = l_ref[...]
    R, T = p.shape
    allv = jnp.maximum(p, l)
    prev = jnp.concatenate([jnp.zeros((R, 1), jnp.float32), allv[:, :-1]], axis=1)
    is_start = allv * (1.0 - prev)
    is_endb = prev * (1.0 - allv)

    t0 = jax.lax.broadcasted_iota(jnp.int32, (T, T), 0)
    t1 = jax.lax.broadcasted_iota(jnp.int32, (T, T), 1)
    tri = (t0 <= t1).astype(jnp.float32)
    P = jax.lax.dot(p, tri, preferred_element_type=jnp.float32)
    L = jax.lax.dot(l, tri, preferred_element_type=jnp.float32)

    mP = jnp.where(is_start > 0, P - p, -1.0)
    mL = jnp.where(is_start > 0, L - l, -1.0)
    k = 1
    while k < T:
        pad = jnp.full((R, k), -1.0, jnp.float32)
        mP = jnp.maximum(mP, jnp.concatenate([pad, mP[:, :-k]], axis=1))
        mL = jnp.maximum(mL, jnp.concatenate([pad, mL[:, :-k]], axis=1))
        k *= 2

    ps = P - mP
    ls = L - mL
    ratio = ps / (ls + 1e-7)
    in_rng = jnp.logical_and(ratio >= 0.7, ratio < 1.0 / 0.7)
    tp_lane = jnp.where(jnp.logical_and(is_endb > 0, in_rng), 1.0, 0.0)
    tp_row = jnp.sum(tp_lane, axis=1, keepdims=True)
    fin = allv[:, T - 1:T]
    rf = (P[:, T - 1:T] - mP[:, T - 1:T]) / (L[:, T - 1:T] - mL[:, T - 1:T] + 1e-7)
    in_f = jnp.logical_and(rf >= 0.7, rf < 1.0 / 0.7)
    tp_row = tp_row + jnp.where(jnp.logical_and(fin > 0, in_f), 1.0, 0.0)
    cnt_row = jnp.sum(is_start, axis=1, keepdims=True)

    n_clip = R // 10
    cb = jax.lax.broadcasted_iota(jnp.int32, (n_clip, R), 0)
    cr = jax.lax.broadcasted_iota(jnp.int32, (n_clip, R), 1)
    onehot = (cr // 10 == cb).astype(jnp.float32)
    tp_clip = jax.lax.dot(onehot, tp_row, preferred_element_type=jnp.float32)
    cnt_clip = jax.lax.dot(onehot, cnt_row, preferred_element_type=jnp.float32)
    denom = 0.5 * tp_clip + 0.5 * cnt_clip
    f = jnp.where(denom > 0, tp_clip / denom, 0.0)
    out_ref[...] = jnp.sum(f, axis=(0, 1), keepdims=True) / n_clip


@jax.jit
def kernel(strong_preds, ground_truths):
    bsz, num_cls, T = strong_preds.shape
    p = strong_preds.reshape(bsz * num_cls, T)
    l = ground_truths.reshape(bsz * num_cls, T)
    out = pl.pallas_call(
        _sed_f1_kernel,
        out_shape=jax.ShapeDtypeStruct((1, 1), jnp.float32),
    )(p, l)
    return out[0, 0]
